# baseline (speedup 1.0000x reference)
import jax
import jax.numpy as jnp
from jax import lax
from jax.experimental import pallas as pl
from jax.experimental.pallas import tpu as pltpu
from jax.experimental.pallas import tpu_sc as plsc

_Q = 8
_V = 1000
_D = 64
_B = 1024
_T = 50

_NC = 2
_NS = 16
_NW = _NC * _NS
_REPS = _NW // _Q
_BPW = _B // _REPS
_LANES = 16
_NSEG = _D // _LANES


def _sc_body(seqT_hbm, tab_hbm, out_hbm, tab_v, idx_v, rows0, rows1,
             sem_o0, sem_o1, sem_i):
    rows = (rows0, rows1)
    sem_o = (sem_o0, sem_o1)

    s = lax.axis_index("s")
    c = lax.axis_index("c")
    q = lax.bitwise_and(s, _Q - 1)
    rep = lax.shift_right_logical(s, 3) * _NC + c
    b0 = rep * _BPW

    pltpu.sync_copy(tab_hbm.at[q], tab_v)
    pltpu.async_copy(seqT_hbm.at[q, pl.ds(b0, _BPW)], idx_v, sem_i).wait()

    def row_block(bl, buf):
        for st, lane_lo in ((0, 0), (16, 0), (32, 0), (34, 14)):
            v = idx_v[bl, pl.ds(st, _LANES)]
            for ln in range(lane_lo, _LANES):
                t = st + ln
                r = v[ln]
                for k in range(_NSEG):
                    rows[buf][t, pl.ds(k * _LANES, _LANES)] = (
                        tab_v[r, pl.ds(k * _LANES, _LANES)])

    def out_slice(bl):
        start = ((b0 + bl) * _Q + q) * _T
        return out_hbm.at[pl.ds(start, _T)]

    row_block(0, 0)
    pltpu.async_copy(rows[0], out_slice(0), sem_o[0])
    row_block(1, 1)
    pltpu.async_copy(rows[1], out_slice(1), sem_o[1])

    def body(it, carry):
        bl = it * 2
        pltpu.make_async_copy(rows[0], out_slice(bl), sem_o[0]).wait()
        row_block(bl, 0)
        pltpu.async_copy(rows[0], out_slice(bl), sem_o[0])
        pltpu.make_async_copy(rows[1], out_slice(bl + 1), sem_o[1]).wait()
        row_block(bl + 1, 1)
        pltpu.async_copy(rows[1], out_slice(bl + 1), sem_o[1])
        return carry

    lax.fori_loop(1, _BPW // 2, body, 0)
    pltpu.make_async_copy(rows[0], out_slice(0), sem_o[0]).wait()
    pltpu.make_async_copy(rows[1], out_slice(1), sem_o[1]).wait()


@jax.jit
def kernel(sequence, tables):
    seq_t = jnp.transpose(sequence, (1, 0, 2)).astype(jnp.int32)
    out_shape = jax.ShapeDtypeStruct((_B * _Q * _T, _D), jnp.float32)
    mesh = plsc.VectorSubcoreMesh(core_axis_name="c", subcore_axis_name="s")
    call = pl.kernel(
        _sc_body,
        mesh=mesh,
        out_type=out_shape,
        scratch_types=[
            pltpu.VMEM((_V, _D), jnp.float32),
            pltpu.VMEM((_BPW, _T), jnp.int32),
            pltpu.VMEM((_T, _D), jnp.float32),
            pltpu.VMEM((_T, _D), jnp.float32),
            pltpu.SemaphoreType.DMA,
            pltpu.SemaphoreType.DMA,
            pltpu.SemaphoreType.DMA,
        ],
        compiler_params=pltpu.CompilerParams(use_tc_tiling_on_sc=False),
    )
    out = call(seq_t, tables)
    return out.reshape(_B, _Q, _T, _D)
